# baseline (speedup 1.0000x reference)
import jax
import jax.numpy as jnp
from jax.experimental import pallas as pl
from jax.experimental.pallas import tpu as pltpu

_N = 1024
_D = 128
_K = 100000
_KB = 2048
_C = 32
_NSTEPS = (_K + _KB - 1) // _KB
_BIG = 3.0e38


def _knn_min_kernel(params_ref, fw_ref, f_ref, mb_ref, out_ref, acc_ref):
    k = pl.program_id(0)
    fw = fw_ref[...]

    def partial_mins(masked):
        pm = None
        for j in range(_KB // _C):
            mbj = mb_ref[pl.ds(j * _C, _C), :]
            m2 = jnp.sum(mbj * mbj, axis=1, keepdims=True)
            s = jax.lax.dot_general(
                mbj.astype(jnp.bfloat16), fw, (((1,), (1,)), ((), ())),
                preferred_element_type=jnp.float32)
            part = s + m2
            if masked:
                row = (k * _KB + j * _C
                       + jax.lax.broadcasted_iota(jnp.int32, (_C, 1), 0))
                part = jnp.where(row < _K, part, _BIG)
            pm = part if pm is None else jnp.minimum(pm, part)
        return pm

    @pl.when(k == 0)
    def _():
        acc_ref[...] = jnp.full((_C, _N), _BIG, jnp.float32)

    @pl.when(k < _NSTEPS - 1)
    def _():
        acc_ref[...] = jnp.minimum(acc_ref[...], partial_mins(False))

    @pl.when(k == _NSTEPS - 1)
    def _():
        acc = jnp.minimum(acc_ref[...], partial_mins(True))
        dmin = jnp.min(acc, axis=0, keepdims=True)
        f = f_ref[...]
        f2 = jax.lax.dot_general(
            jnp.ones((1, _D), jnp.float32), f * f, (((1,), (1,)), ((), ())),
            preferred_element_type=jnp.float32)
        d2 = jnp.maximum(dmin + f2, 1e-12)
        d = jnp.sqrt(d2)
        mn = params_ref[0, 0]
        mx = params_ref[0, 1]
        out_ref[...] = (d - mn) / (mx - mn)


def kernel(features, memory_bank, min_val, max_val):
    params = jnp.stack([min_val, max_val]).reshape(1, 2)
    fw = (features * -2.0).astype(jnp.bfloat16)
    out = pl.pallas_call(
        _knn_min_kernel,
        grid=(_NSTEPS,),
        in_specs=[
            pl.BlockSpec(memory_space=pltpu.SMEM),
            pl.BlockSpec((_N, _D), lambda k: (0, 0)),
            pl.BlockSpec((_N, _D), lambda k: (0, 0)),
            pl.BlockSpec((_KB, _D), lambda k: (k, 0)),
        ],
        out_specs=pl.BlockSpec((1, _N), lambda k: (0, 0)),
        out_shape=jax.ShapeDtypeStruct((1, _N), jnp.float32),
        scratch_shapes=[pltpu.VMEM((_C, _N), jnp.float32)],
        compiler_params=pltpu.CompilerParams(
            dimension_semantics=("arbitrary",)),
    )(params, fw, features, memory_bank)
    return out.reshape(_N)


# speedup vs baseline: 3.0670x; 3.0670x over previous
import jax
import jax.numpy as jnp
from jax.experimental import pallas as pl
from jax.experimental.pallas import tpu as pltpu

_N = 1024
_QB = 256
_D = 128
_K = 100000
_KB = 2048
_NCH = _KB // 128
_NSTEPS = (_K + _KB - 1) // _KB
_BIG = 3.0e38


def _knn_min_kernel(params_ref, fw_ref, f_ref, mb_ref, out_ref,
                    mbb_ref, m2_ref, acc_ref):
    k = pl.program_id(0)
    ones_row = jnp.ones((1, _D), jnp.float32)

    for j in range(_NCH):
        mbj = mb_ref[pl.ds(j * 128, 128), :]
        mbb_ref[pl.ds(j * 128, 128), :] = mbj.astype(jnp.bfloat16)
        m2_ref[0:1, pl.ds(j * 128, 128)] = jax.lax.dot_general(
            ones_row, mbj * mbj, (((1,), (1,)), ((), ())),
            preferred_element_type=jnp.float32)

    def qblock_min(q, masked):
        fq = fw_ref[pl.ds(q * _QB, _QB), :]
        pm = None
        for j in range(_NCH):
            mbjb = mbb_ref[pl.ds(j * 128, 128), :]
            s = jax.lax.dot_general(
                fq, mbjb, (((1,), (1,)), ((), ())),
                preferred_element_type=jnp.float32)
            part = s + m2_ref[0:1, pl.ds(j * 128, 128)]
            if masked:
                col = (k * _KB + j * 128
                       + jax.lax.broadcasted_iota(jnp.int32, (1, 128), 1))
                part = jnp.where(col < _K, part, _BIG)
            pm = part if pm is None else jnp.minimum(pm, part)
        return pm

    @pl.when(k == 0)
    def _():
        acc_ref[...] = jnp.full((_N, 128), _BIG, jnp.float32)

    @pl.when(k < _NSTEPS - 1)
    def _():
        for q in range(_N // _QB):
            sl = pl.ds(q * _QB, _QB)
            acc_ref[sl, :] = jnp.minimum(acc_ref[sl, :], qblock_min(q, False))

    @pl.when(k == _NSTEPS - 1)
    def _():
        for q in range(_N // _QB):
            sl = pl.ds(q * _QB, _QB)
            acc_ref[sl, :] = jnp.minimum(acc_ref[sl, :], qblock_min(q, True))
        f = f_ref[...]
        f2 = jnp.sum(f * f, axis=1, keepdims=True)
        d2 = jnp.maximum(jnp.min(acc_ref[...], axis=1, keepdims=True) + f2,
                         1e-12)
        d = jnp.sqrt(d2)
        mn = params_ref[0, 0]
        mx = params_ref[0, 1]
        out_ref[...] = (d - mn) / (mx - mn)


def kernel(features, memory_bank, min_val, max_val):
    params = jnp.stack([min_val, max_val]).reshape(1, 2)
    fw = (features * -2.0).astype(jnp.bfloat16)
    out = pl.pallas_call(
        _knn_min_kernel,
        grid=(_NSTEPS,),
        in_specs=[
            pl.BlockSpec(memory_space=pltpu.SMEM),
            pl.BlockSpec((_N, _D), lambda k: (0, 0)),
            pl.BlockSpec((_N, _D), lambda k: (0, 0)),
            pl.BlockSpec((_KB, _D), lambda k: (k, 0)),
        ],
        out_specs=pl.BlockSpec((_N, 1), lambda k: (0, 0)),
        out_shape=jax.ShapeDtypeStruct((_N, 1), jnp.float32),
        scratch_shapes=[
            pltpu.VMEM((_KB, _D), jnp.bfloat16),
            pltpu.VMEM((8, _KB), jnp.float32),
            pltpu.VMEM((_N, 128), jnp.float32),
        ],
        compiler_params=pltpu.CompilerParams(
            dimension_semantics=("arbitrary",)),
    )(params, fw, features, memory_bank)
    return out.reshape(_N)


# speedup vs baseline: 3.1898x; 1.0400x over previous
import jax
import jax.numpy as jnp
from jax.experimental import pallas as pl
from jax.experimental.pallas import tpu as pltpu

_N = 1024
_QB = 128
_D = 128
_K = 100000
_KB = 2048
_TB = 256
_NCH = _KB // _TB
_NSTEPS = (_K + _KB - 1) // _KB
_BIG = 3.0e38


def _knn_min_kernel(params_ref, fw_ref, f_ref, mb_ref, out_ref,
                    mbb_ref, m2_ref, acc_ref):
    k = pl.program_id(0)
    ones_row = jnp.ones((1, _D), jnp.float32)

    for j in range(_NCH):
        mbj = mb_ref[pl.ds(j * _TB, _TB), :]
        mbb_ref[pl.ds(j * _TB, _TB), :] = mbj.astype(jnp.bfloat16)
        m2_ref[0:1, pl.ds(j * _TB, _TB)] = jax.lax.dot_general(
            ones_row, mbj * mbj, (((1,), (1,)), ((), ())),
            preferred_element_type=jnp.float32)

    def qblock_min(q, masked):
        fq = fw_ref[pl.ds(q * _QB, _QB), :]
        pm = None
        for j in range(_NCH):
            mbjb = mbb_ref[pl.ds(j * _TB, _TB), :]
            s = jax.lax.dot_general(
                fq, mbjb, (((1,), (1,)), ((), ())),
                preferred_element_type=jnp.float32)
            part = s + m2_ref[0:1, pl.ds(j * _TB, _TB)]
            if masked:
                col = (k * _KB + j * _TB
                       + jax.lax.broadcasted_iota(jnp.int32, (1, _TB), 1))
                part = jnp.where(col < _K, part, _BIG)
            for h in range(_TB // 128):
                ph = part[:, h * 128:(h + 1) * 128]
                pm = ph if pm is None else jnp.minimum(pm, ph)
        return pm

    @pl.when(k == 0)
    def _():
        acc_ref[...] = jnp.full((_N, 128), _BIG, jnp.float32)

    @pl.when(k < _NSTEPS - 1)
    def _():
        for q in range(_N // _QB):
            sl = pl.ds(q * _QB, _QB)
            acc_ref[sl, :] = jnp.minimum(acc_ref[sl, :], qblock_min(q, False))

    @pl.when(k == _NSTEPS - 1)
    def _():
        for q in range(_N // _QB):
            sl = pl.ds(q * _QB, _QB)
            acc_ref[sl, :] = jnp.minimum(acc_ref[sl, :], qblock_min(q, True))
        f = f_ref[...]
        f2 = jnp.sum(f * f, axis=1, keepdims=True)
        d2 = jnp.maximum(jnp.min(acc_ref[...], axis=1, keepdims=True) + f2,
                         1e-12)
        d = jnp.sqrt(d2)
        mn = params_ref[0, 0]
        mx = params_ref[0, 1]
        out_ref[...] = (d - mn) / (mx - mn)


def kernel(features, memory_bank, min_val, max_val):
    params = jnp.stack([min_val, max_val]).reshape(1, 2)
    fw = (features * -2.0).astype(jnp.bfloat16)
    out = pl.pallas_call(
        _knn_min_kernel,
        grid=(_NSTEPS,),
        in_specs=[
            pl.BlockSpec(memory_space=pltpu.SMEM),
            pl.BlockSpec((_N, _D), lambda k: (0, 0)),
            pl.BlockSpec((_N, _D), lambda k: (0, 0)),
            pl.BlockSpec((_KB, _D), lambda k: (k, 0)),
        ],
        out_specs=pl.BlockSpec((_N, 1), lambda k: (0, 0)),
        out_shape=jax.ShapeDtypeStruct((_N, 1), jnp.float32),
        scratch_shapes=[
            pltpu.VMEM((_KB, _D), jnp.bfloat16),
            pltpu.VMEM((8, _KB), jnp.float32),
            pltpu.VMEM((_N, 128), jnp.float32),
        ],
        compiler_params=pltpu.CompilerParams(
            dimension_semantics=("arbitrary",)),
    )(params, fw, features, memory_bank)
    return out.reshape(_N)


# speedup vs baseline: 3.2412x; 1.0161x over previous
import jax
import jax.numpy as jnp
from jax.experimental import pallas as pl
from jax.experimental.pallas import tpu as pltpu

_N = 1024
_QB = 128
_D = 128
_K = 100000
_KB = 2048
_TB = 256
_NCH = _KB // _TB
_NSTEPS = (_K + _KB - 1) // _KB
_BIG = 3.0e38


def _knn_min_kernel(params_ref, fw_ref, f_ref, mb_ref, out_ref,
                    mbb_ref, m2_ref, acc_ref):
    k = pl.program_id(0)
    ones_row = jnp.ones((1, _D), jnp.float32)

    for j in range(_NCH):
        mbj = mb_ref[pl.ds(j * _TB, _TB), :]
        mbb_ref[pl.ds(j * _TB, _TB), :] = mbj.astype(jnp.bfloat16)
        m2_ref[0:1, pl.ds(j * _TB, _TB)] = jax.lax.dot_general(
            ones_row, mbj * mbj, (((1,), (1,)), ((), ())),
            preferred_element_type=jnp.float32)

    def qblock_min(q, masked):
        fq = fw_ref[:, pl.ds(q * _QB, _QB)]
        pm = None
        for j in range(_NCH):
            mbjb = mbb_ref[pl.ds(j * _TB, _TB), :]
            s = jax.lax.dot_general(
                fq, mbjb, (((0,), (1,)), ((), ())),
                preferred_element_type=jnp.float32)
            part = s + m2_ref[0:1, pl.ds(j * _TB, _TB)]
            if masked:
                col = (k * _KB + j * _TB
                       + jax.lax.broadcasted_iota(jnp.int32, (1, _TB), 1))
                part = jnp.where(col < _K, part, _BIG)
            for h in range(_TB // 128):
                ph = part[:, h * 128:(h + 1) * 128]
                pm = ph if pm is None else jnp.minimum(pm, ph)
        return pm

    @pl.when(k == 0)
    def _():
        acc_ref[...] = jnp.full((_N, 128), _BIG, jnp.float32)

    @pl.when(k < _NSTEPS - 1)
    def _():
        for q in range(_N // _QB):
            sl = pl.ds(q * _QB, _QB)
            acc_ref[sl, :] = jnp.minimum(acc_ref[sl, :], qblock_min(q, False))

    @pl.when(k == _NSTEPS - 1)
    def _():
        for q in range(_N // _QB):
            sl = pl.ds(q * _QB, _QB)
            acc_ref[sl, :] = jnp.minimum(acc_ref[sl, :], qblock_min(q, True))
        f = f_ref[...]
        f2 = jnp.sum(f * f, axis=1, keepdims=True)
        dmin = jnp.min(acc_ref[...], axis=1, keepdims=True)
        d2 = jnp.maximum(dmin + f2, 1e-12)
        d = jnp.sqrt(d2)
        mn = params_ref[0, 0]
        mx = params_ref[0, 1]
        out_ref[...] = (d - mn) / (mx - mn)


def kernel(features, memory_bank, min_val, max_val):
    params = jnp.stack([min_val, max_val]).reshape(1, 2)
    fw = (features * -2.0).astype(jnp.bfloat16).T
    out = pl.pallas_call(
        _knn_min_kernel,
        grid=(_NSTEPS,),
        in_specs=[
            pl.BlockSpec(memory_space=pltpu.SMEM),
            pl.BlockSpec((_D, _N), lambda k: (0, 0)),
            pl.BlockSpec((_N, _D), lambda k: (0, 0)),
            pl.BlockSpec((_KB, _D), lambda k: (k, 0)),
        ],
        out_specs=pl.BlockSpec((_N, 1), lambda k: (0, 0)),
        out_shape=jax.ShapeDtypeStruct((_N, 1), jnp.float32),
        scratch_shapes=[
            pltpu.VMEM((_KB, _D), jnp.bfloat16),
            pltpu.VMEM((8, _KB), jnp.float32),
            pltpu.VMEM((_N, 128), jnp.float32),
        ],
        compiler_params=pltpu.CompilerParams(
            dimension_semantics=("arbitrary",)),
    )(params, fw, features, memory_bank)
    return out.reshape(_N)


# speedup vs baseline: 4.2688x; 1.3170x over previous
import jax
import jax.numpy as jnp
from jax.experimental import pallas as pl
from jax.experimental.pallas import tpu as pltpu

_N = 1024
_QB = 128
_D = 128
_K = 100000
_KB = 2048
_TB = 256
_NCH = _KB // _TB
_NSTEPS = (_K + _KB - 1) // _KB
_BIG = 3.0e38


def _knn_min_kernel(params_ref, fw_ref, f_ref, mb_ref, out_ref,
                    mbb_ref, m2_ref, acc_ref):
    k = pl.program_id(0)
    ones_row = jnp.ones((1, _D), jnp.float32)

    for j in range(_NCH):
        mbj = mb_ref[pl.ds(j * _TB, _TB), :]
        mbb_ref[pl.ds(j * _TB, _TB), :] = mbj.astype(jnp.float8_e4m3fn)
        m2_ref[0:1, pl.ds(j * _TB, _TB)] = jax.lax.dot_general(
            ones_row, mbj * mbj, (((1,), (1,)), ((), ())),
            preferred_element_type=jnp.float32)

    def qblock_min(q, masked):
        fq = fw_ref[:, pl.ds(q * _QB, _QB)]
        pm = None
        for j in range(_NCH):
            mbjb = mbb_ref[pl.ds(j * _TB, _TB), :]
            s = jax.lax.dot_general(
                fq, mbjb, (((0,), (1,)), ((), ())),
                preferred_element_type=jnp.float32)
            part = s + m2_ref[0:1, pl.ds(j * _TB, _TB)]
            if masked:
                col = (k * _KB + j * _TB
                       + jax.lax.broadcasted_iota(jnp.int32, (1, _TB), 1))
                part = jnp.where(col < _K, part, _BIG)
            for h in range(_TB // 128):
                ph = part[:, h * 128:(h + 1) * 128]
                pm = ph if pm is None else jnp.minimum(pm, ph)
        return pm

    @pl.when(k == 0)
    def _():
        acc_ref[...] = jnp.full((_N, 128), _BIG, jnp.float32)

    @pl.when(k < _NSTEPS - 1)
    def _():
        for q in range(_N // _QB):
            sl = pl.ds(q * _QB, _QB)
            acc_ref[sl, :] = jnp.minimum(acc_ref[sl, :], qblock_min(q, False))

    @pl.when(k == _NSTEPS - 1)
    def _():
        for q in range(_N // _QB):
            sl = pl.ds(q * _QB, _QB)
            acc_ref[sl, :] = jnp.minimum(acc_ref[sl, :], qblock_min(q, True))
        f = f_ref[...]
        f2 = jnp.sum(f * f, axis=1, keepdims=True)
        dmin = jnp.min(acc_ref[...], axis=1, keepdims=True)
        d2 = jnp.maximum(dmin + f2, 1e-12)
        d = jnp.sqrt(d2)
        mn = params_ref[0, 0]
        mx = params_ref[0, 1]
        out_ref[...] = (d - mn) / (mx - mn)


def kernel(features, memory_bank, min_val, max_val):
    params = jnp.stack([min_val, max_val]).reshape(1, 2)
    fw = (features * -2.0).astype(jnp.float8_e4m3fn).T
    out = pl.pallas_call(
        _knn_min_kernel,
        grid=(_NSTEPS,),
        in_specs=[
            pl.BlockSpec(memory_space=pltpu.SMEM),
            pl.BlockSpec((_D, _N), lambda k: (0, 0)),
            pl.BlockSpec((_N, _D), lambda k: (0, 0)),
            pl.BlockSpec((_KB, _D), lambda k: (k, 0)),
        ],
        out_specs=pl.BlockSpec((_N, 1), lambda k: (0, 0)),
        out_shape=jax.ShapeDtypeStruct((_N, 1), jnp.float32),
        scratch_shapes=[
            pltpu.VMEM((_KB, _D), jnp.float8_e4m3fn),
            pltpu.VMEM((8, _KB), jnp.float32),
            pltpu.VMEM((_N, 128), jnp.float32),
        ],
        compiler_params=pltpu.CompilerParams(
            dimension_semantics=("arbitrary",)),
    )(params, fw, features, memory_bank)
    return out.reshape(_N)


# speedup vs baseline: 4.3149x; 1.0108x over previous
import jax
import jax.numpy as jnp
from jax.experimental import pallas as pl
from jax.experimental.pallas import tpu as pltpu

_N = 1024
_QB = 128
_D = 128
_K = 100000
_KB = 2048
_TB = 256
_NCH = _KB // _TB
_NSTEPS = (_K + _KB - 1) // _KB
_BIG = 3.0e38


def _knn_min_kernel(params_ref, fw_ref, f_ref, mb_ref, out_ref,
                    mbb_ref, m2_ref, acc_ref):
    k = pl.program_id(0)
    ones_row = jnp.ones((1, _D), jnp.float32)

    for j in range(_NCH):
        mbj = mb_ref[pl.ds(j * _TB, _TB), :]
        mbb_ref[pl.ds(j * _TB, _TB), :] = mbj.astype(jnp.float8_e4m3fn)
        m2_ref[0:1, pl.ds(j * _TB, _TB)] = jax.lax.dot_general(
            ones_row, mbj * mbj, (((1,), (1,)), ((), ())),
            preferred_element_type=jnp.float32).astype(jnp.bfloat16)

    def qblock_min(q, masked):
        fq = fw_ref[:, pl.ds(q * _QB, _QB)]
        pm = None
        for j in range(_NCH):
            mbjb = mbb_ref[pl.ds(j * _TB, _TB), :]
            s = jax.lax.dot_general(
                fq, mbjb, (((0,), (1,)), ((), ())),
                preferred_element_type=jnp.float32)
            part = (s.astype(jnp.bfloat16)
                    + m2_ref[0:1, pl.ds(j * _TB, _TB)])
            if masked:
                col = (k * _KB + j * _TB
                       + jax.lax.broadcasted_iota(jnp.int32, (1, _TB), 1))
                part = jnp.where(col < _K, part, _BIG)
            for h in range(_TB // 128):
                ph = part[:, h * 128:(h + 1) * 128]
                pm = ph if pm is None else jnp.minimum(pm, ph)
        return pm

    @pl.when(k == 0)
    def _():
        acc_ref[...] = jnp.full((_N, 128), _BIG, jnp.bfloat16)

    @pl.when(k < _NSTEPS - 1)
    def _():
        for q in range(_N // _QB):
            sl = pl.ds(q * _QB, _QB)
            acc_ref[sl, :] = jnp.minimum(acc_ref[sl, :], qblock_min(q, False))

    @pl.when(k == _NSTEPS - 1)
    def _():
        for q in range(_N // _QB):
            sl = pl.ds(q * _QB, _QB)
            acc_ref[sl, :] = jnp.minimum(acc_ref[sl, :], qblock_min(q, True))
        f = f_ref[...]
        f2 = jnp.sum(f * f, axis=1, keepdims=True)
        dmin = jnp.min(acc_ref[...].astype(jnp.float32), axis=1,
                       keepdims=True)
        d2 = jnp.maximum(dmin + f2, 1e-12)
        d = jnp.sqrt(d2)
        mn = params_ref[0, 0]
        mx = params_ref[0, 1]
        out_ref[...] = (d - mn) / (mx - mn)


def kernel(features, memory_bank, min_val, max_val):
    params = jnp.stack([min_val, max_val]).reshape(1, 2)
    fw = (features * -2.0).astype(jnp.float8_e4m3fn).T
    out = pl.pallas_call(
        _knn_min_kernel,
        grid=(_NSTEPS,),
        in_specs=[
            pl.BlockSpec(memory_space=pltpu.SMEM),
            pl.BlockSpec((_D, _N), lambda k: (0, 0)),
            pl.BlockSpec((_N, _D), lambda k: (0, 0)),
            pl.BlockSpec((_KB, _D), lambda k: (k, 0)),
        ],
        out_specs=pl.BlockSpec((_N, 1), lambda k: (0, 0)),
        out_shape=jax.ShapeDtypeStruct((_N, 1), jnp.float32),
        scratch_shapes=[
            pltpu.VMEM((_KB, _D), jnp.float8_e4m3fn),
            pltpu.VMEM((8, _KB), jnp.bfloat16),
            pltpu.VMEM((_N, 128), jnp.bfloat16),
        ],
        compiler_params=pltpu.CompilerParams(
            dimension_semantics=("arbitrary",)),
    )(params, fw, features, memory_bank)
    return out.reshape(_N)


# speedup vs baseline: 5.2732x; 1.2221x over previous
import jax
import jax.numpy as jnp
from jax.experimental import pallas as pl
from jax.experimental.pallas import tpu as pltpu

_N = 1024
_QB = 128
_D = 128
_K = 100000
_KB = 4096
_TB = 256
_NCH = _KB // _TB
_NSTEPS = (_K + _KB - 1) // _KB
_BIG = 3.0e38


def _knn_min_kernel(params_ref, fw_ref, f_ref, mb_ref, out_ref,
                    mbb_ref, m2_ref, acc_ref):
    k = pl.program_id(0)
    ones_row = jnp.ones((1, _D), jnp.float32)

    for j in range(_NCH):
        mbj = mb_ref[pl.ds(j * _TB, _TB), :]
        mbb_ref[pl.ds(j * _TB, _TB), :] = mbj.astype(jnp.float8_e4m3fn)
        m2_ref[0:1, pl.ds(j * _TB, _TB)] = jax.lax.dot_general(
            ones_row, mbj * mbj, (((1,), (1,)), ((), ())),
            preferred_element_type=jnp.float32).astype(jnp.bfloat16)

    def qblock_min(q, masked):
        fq = fw_ref[:, pl.ds(q * _QB, _QB)]
        pm = None
        for j in range(_NCH):
            mbjb = mbb_ref[pl.ds(j * _TB, _TB), :]
            s = jax.lax.dot_general(
                fq, mbjb, (((0,), (1,)), ((), ())),
                preferred_element_type=jnp.float32)
            part = (s.astype(jnp.bfloat16)
                    + m2_ref[0:1, pl.ds(j * _TB, _TB)])
            if masked:
                col = (k * _KB + j * _TB
                       + jax.lax.broadcasted_iota(jnp.int32, (1, _TB), 1))
                part = jnp.where(col < _K, part, _BIG)
            for h in range(_TB // 128):
                ph = part[:, h * 128:(h + 1) * 128]
                pm = ph if pm is None else jnp.minimum(pm, ph)
        return pm

    @pl.when(k == 0)
    def _():
        acc_ref[...] = jnp.full((_N, 128), _BIG, jnp.bfloat16)

    @pl.when(k < _NSTEPS - 1)
    def _():
        for q in range(_N // _QB):
            sl = pl.ds(q * _QB, _QB)
            acc_ref[sl, :] = jnp.minimum(acc_ref[sl, :], qblock_min(q, False))

    @pl.when(k == _NSTEPS - 1)
    def _():
        for q in range(_N // _QB):
            sl = pl.ds(q * _QB, _QB)
            acc_ref[sl, :] = jnp.minimum(acc_ref[sl, :], qblock_min(q, True))
        f = f_ref[...]
        f2 = jnp.sum(f * f, axis=1, keepdims=True)
        dmin = jnp.min(acc_ref[...].astype(jnp.float32), axis=1,
                       keepdims=True)
        d2 = jnp.maximum(dmin + f2, 1e-12)
        d = jnp.sqrt(d2)
        mn = params_ref[0, 0]
        mx = params_ref[0, 1]
        out_ref[...] = (d - mn) / (mx - mn)


def kernel(features, memory_bank, min_val, max_val):
    params = jnp.stack([min_val, max_val]).reshape(1, 2)
    fw = (features * -2.0).astype(jnp.float8_e4m3fn).T
    out = pl.pallas_call(
        _knn_min_kernel,
        grid=(_NSTEPS,),
        in_specs=[
            pl.BlockSpec(memory_space=pltpu.SMEM),
            pl.BlockSpec((_D, _N), lambda k: (0, 0)),
            pl.BlockSpec((_N, _D), lambda k: (0, 0)),
            pl.BlockSpec((_KB, _D), lambda k: (k, 0)),
        ],
        out_specs=pl.BlockSpec((_N, 1), lambda k: (0, 0)),
        out_shape=jax.ShapeDtypeStruct((_N, 1), jnp.float32),
        scratch_shapes=[
            pltpu.VMEM((_KB, _D), jnp.float8_e4m3fn),
            pltpu.VMEM((8, _KB), jnp.bfloat16),
            pltpu.VMEM((_N, 128), jnp.bfloat16),
        ],
        compiler_params=pltpu.CompilerParams(
            dimension_semantics=("arbitrary",)),
    )(params, fw, features, memory_bank)
    return out.reshape(_N)


# speedup vs baseline: 5.9332x; 1.1252x over previous
import jax
import jax.numpy as jnp
from jax.experimental import pallas as pl
from jax.experimental.pallas import tpu as pltpu

_N = 1024
_QB = 128
_D = 128
_K = 100000
_KB = 12800
_TB = 256
_NCH = _KB // _TB
_NSTEPS = (_K + _KB - 1) // _KB
_BIG = 3.0e38


def _knn_min_kernel(params_ref, fw_ref, f_ref, mb_ref, out_ref,
                    mbb_ref, m2_ref, acc_ref):
    k = pl.program_id(0)
    ones_row = jnp.ones((1, _D), jnp.float32)

    for j in range(_NCH):
        mbj = mb_ref[pl.ds(j * _TB, _TB), :]
        mbb_ref[pl.ds(j * _TB, _TB), :] = mbj.astype(jnp.float8_e4m3fn)
        m2_ref[0:1, pl.ds(j * _TB, _TB)] = jax.lax.dot_general(
            ones_row, mbj * mbj, (((1,), (1,)), ((), ())),
            preferred_element_type=jnp.float32).astype(jnp.bfloat16)

    def qblock_min(q, masked):
        fq = fw_ref[:, pl.ds(q * _QB, _QB)]
        pm = None
        for j in range(_NCH):
            mbjb = mbb_ref[pl.ds(j * _TB, _TB), :]
            s = jax.lax.dot_general(
                fq, mbjb, (((0,), (1,)), ((), ())),
                preferred_element_type=jnp.float32)
            part = (s.astype(jnp.bfloat16)
                    + m2_ref[0:1, pl.ds(j * _TB, _TB)])
            if masked:
                col = (k * _KB + j * _TB
                       + jax.lax.broadcasted_iota(jnp.int32, (1, _TB), 1))
                part = jnp.where(col < _K, part, _BIG)
            for h in range(_TB // 128):
                ph = part[:, h * 128:(h + 1) * 128]
                pm = ph if pm is None else jnp.minimum(pm, ph)
        return pm

    @pl.when(k == 0)
    def _():
        acc_ref[...] = jnp.full((_N, 128), _BIG, jnp.bfloat16)

    @pl.when(k < _NSTEPS - 1)
    def _():
        for q in range(_N // _QB):
            sl = pl.ds(q * _QB, _QB)
            acc_ref[sl, :] = jnp.minimum(acc_ref[sl, :], qblock_min(q, False))

    @pl.when(k == _NSTEPS - 1)
    def _():
        for q in range(_N // _QB):
            sl = pl.ds(q * _QB, _QB)
            acc_ref[sl, :] = jnp.minimum(acc_ref[sl, :], qblock_min(q, True))
        f = f_ref[...]
        f2 = jnp.sum(f * f, axis=1, keepdims=True)
        dmin = jnp.min(acc_ref[...].astype(jnp.float32), axis=1,
                       keepdims=True)
        d2 = jnp.maximum(dmin + f2, 1e-12)
        d = jnp.sqrt(d2)
        mn = params_ref[0, 0]
        mx = params_ref[0, 1]
        out_ref[...] = (d - mn) / (mx - mn)


def kernel(features, memory_bank, min_val, max_val):
    params = jnp.stack([min_val, max_val]).reshape(1, 2)
    fw = (features * -2.0).astype(jnp.float8_e4m3fn).T
    out = pl.pallas_call(
        _knn_min_kernel,
        grid=(_NSTEPS,),
        in_specs=[
            pl.BlockSpec(memory_space=pltpu.SMEM),
            pl.BlockSpec((_D, _N), lambda k: (0, 0)),
            pl.BlockSpec((_N, _D), lambda k: (0, 0)),
            pl.BlockSpec((_KB, _D), lambda k: (k, 0)),
        ],
        out_specs=pl.BlockSpec((_N, 1), lambda k: (0, 0)),
        out_shape=jax.ShapeDtypeStruct((_N, 1), jnp.float32),
        scratch_shapes=[
            pltpu.VMEM((_KB, _D), jnp.float8_e4m3fn),
            pltpu.VMEM((8, _KB), jnp.bfloat16),
            pltpu.VMEM((_N, 128), jnp.bfloat16),
        ],
        compiler_params=pltpu.CompilerParams(
            dimension_semantics=("arbitrary",)),
    )(params, fw, features, memory_bank)
    return out.reshape(_N)
